# Initial kernel scaffold; baseline (speedup 1.0000x reference)
import jax
import jax.numpy as jnp
from jax.experimental import pallas as pl


def kernel(token_ids, weight):
    raise NotImplementedError("write your pallas kernel here")


# speedup vs baseline: 1.4766x; 1.4766x over previous
import functools

import jax
import jax.numpy as jnp
from jax import lax
from jax.experimental import pallas as pl
from jax.experimental.pallas import tpu as pltpu
from jax.experimental.pallas import tpu_sc as plsc

_D = 32
_B = 4096 * 200
_NW = 32
_BPW = _B // _NW
_CHUNK = 1600
_NCHUNK = _BPW // _CHUNK


def _make_gather():
    mesh = plsc.VectorSubcoreMesh(core_axis_name="c", subcore_axis_name="s")

    @functools.partial(
        pl.kernel,
        mesh=mesh,
        out_type=jax.ShapeDtypeStruct((_B, _D), jnp.float32),
        compiler_params=pltpu.CompilerParams(use_tc_tiling_on_sc=False),
        scratch_types=[
            pltpu.VMEM((_CHUNK,), jnp.int32),
            pltpu.VMEM((_CHUNK, _D), jnp.float32),
            pltpu.SemaphoreType.DMA,
        ],
    )
    def gather_kernel(idx_hbm, table_hbm, out_hbm, idx_v, rows_v, sem):
        wid = lax.axis_index("s") * 2 + lax.axis_index("c")
        base = wid * _BPW

        def body(i, carry):
            off = base + i * _CHUNK
            pltpu.sync_copy(idx_hbm.at[pl.ds(off, _CHUNK)], idx_v)
            pltpu.async_copy(table_hbm.at[idx_v], rows_v, sem).wait()
            pltpu.sync_copy(rows_v, out_hbm.at[pl.ds(off, _CHUNK)])
            return carry

        lax.fori_loop(0, _NCHUNK, body, 0)

    return gather_kernel


_gather = _make_gather()


def kernel(token_ids, weight):
    idx = token_ids.reshape(-1).astype(jnp.int32)
    out = _gather(idx, weight)
    return out.reshape(*token_ids.shape, _D)


# speedup vs baseline: 1.4927x; 1.0109x over previous
import functools

import jax
import jax.numpy as jnp
from jax import lax
from jax.experimental import pallas as pl
from jax.experimental.pallas import tpu as pltpu
from jax.experimental.pallas import tpu_sc as plsc

_D = 32
_B = 4096 * 200
_NW = 32
_BPW = _B // _NW
_CHUNK = 1600
_NCHUNK = _BPW // _CHUNK


def _make_gather():
    mesh = plsc.VectorSubcoreMesh(core_axis_name="c", subcore_axis_name="s")

    @functools.partial(
        pl.kernel,
        mesh=mesh,
        out_type=jax.ShapeDtypeStruct((_B, _D), jnp.float32),
        compiler_params=pltpu.CompilerParams(use_tc_tiling_on_sc=False),
        scratch_types=[
            pltpu.VMEM((2, _CHUNK), jnp.int32),
            pltpu.VMEM((2, _CHUNK, _D), jnp.float32),
            pltpu.SemaphoreType.DMA,
            pltpu.SemaphoreType.DMA,
            pltpu.SemaphoreType.DMA,
            pltpu.SemaphoreType.DMA,
            pltpu.SemaphoreType.DMA,
            pltpu.SemaphoreType.DMA,
        ],
    )
    def gather_kernel(idx_hbm, table_hbm, out_hbm, idx_v, rows_v,
                      si0, si1, sg0, sg1, sw0, sw1):
        wid = lax.axis_index("s") * 2 + lax.axis_index("c")
        base = wid * _BPW
        si, sg, sw = (si0, si1), (sg0, sg1), (sw0, sw1)

        def idx_copy(i, p):
            off = base + i * _CHUNK
            return pltpu.async_copy(
                idx_hbm.at[pl.ds(off, _CHUNK)], idx_v.at[p], si[p])

        def gather(i, p):
            return pltpu.async_copy(
                table_hbm.at[idx_v.at[p]], rows_v.at[p], sg[p])

        def write(i, p):
            off = base + i * _CHUNK
            return pltpu.async_copy(
                rows_v.at[p], out_hbm.at[pl.ds(off, _CHUNK)], sw[p])

        h_idx = [None] * _NCHUNK
        h_w = [None] * _NCHUNK
        h_idx[0] = idx_copy(0, 0)
        if _NCHUNK > 1:
            h_idx[1] = idx_copy(1, 1)
        for i in range(_NCHUNK):
            p = i % 2
            h_idx[i].wait()
            if i >= 2:
                h_w[i - 2].wait()
            g = gather(i, p)
            g.wait()
            h_w[i] = write(i, p)
            if i + 2 < _NCHUNK:
                h_idx[i + 2] = idx_copy(i + 2, p)
        h_w[_NCHUNK - 2].wait()
        h_w[_NCHUNK - 1].wait()

    return gather_kernel


_gather = _make_gather()


def kernel(token_ids, weight):
    idx = token_ids.reshape(-1).astype(jnp.int32)
    out = _gather(idx, weight)
    return out.reshape(*token_ids.shape, _D)
